# baseline (speedup 1.0000x reference)
import functools

import jax
import jax.numpy as jnp
from jax import lax
from jax.experimental import pallas as pl
from jax.experimental.pallas import tpu as pltpu
from jax.experimental.pallas import tpu_sc as plsc

N_VOCAB = 1000
EMB = 64
B = 16384
KP1 = 11
NPAIR = KP1 * B
NROWS = NPAIR // 128
GBLK = 8

_info = plsc.get_sparse_core_info()
_NC, _NS = _info.num_cores, _info.num_subcores
NW = _NC * _NS
RPW = NROWS // NW


def _prep_body(vt_ref, ut_ref, c_ref, xt_ref, g_ref, idx_ref):
    vt = vt_ref[...]
    for t in range(GBLK):
        hi = min(N_VOCAB, (t + 1) * 128)
        gt = lax.dot_general(
            vt, ut_ref[:, t * 128:hi],
            dimension_numbers=(((0,), (0,)), ((), ())),
            preferred_element_type=jnp.float32)
        if hi - t * 128 < 128:
            gt = jnp.concatenate(
                [gt, jnp.zeros((N_VOCAB, 128 - (hi - t * 128)), jnp.float32)],
                axis=1)
        g_ref[t * N_VOCAB:(t + 1) * N_VOCAB, :] = gt
    xt = xt_ref[...]
    c = c_ref[...]
    y = (xt >> 7) * 128000 + c[None, :] * 128 + (xt & 127)
    idx_ref[...] = y.reshape(NROWS, 128)


def _softmax_body(x_ref, o_ref):
    x = x_ref[...]
    blocks = [x[k * 128:(k + 1) * 128, :] for k in range(KP1)]
    m = blocks[0]
    for bl in blocks[1:]:
        m = jnp.maximum(m, bl)
    s = jnp.zeros_like(m)
    for bl in blocks:
        s = s + jnp.exp(bl - m)
    per = m + jnp.log(s) - blocks[0]
    o_ref[0, 0] = jnp.sum(per) / B


_mesh = plsc.VectorSubcoreMesh(core_axis_name="c", subcore_axis_name="s")


@functools.partial(
    pl.kernel,
    mesh=_mesh,
    compiler_params=pltpu.CompilerParams(use_tc_tiling_on_sc=False),
    out_type=jax.ShapeDtypeStruct((NROWS, 128), jnp.float32),
    scratch_types=[
        pltpu.VMEM((RPW, 128), jnp.int32),
        pltpu.VMEM((RPW, 128), jnp.float32),
        pltpu.VMEM_SHARED((GBLK * N_VOCAB * 128,), jnp.float32),
        pltpu.SemaphoreType.DMA,
    ],
)
def _gather_sc(idx_hbm, g_hbm, out_hbm, ridx, dest, gsh, sem_g):
    wid = lax.axis_index("s") * _NC + lax.axis_index("c")
    base = wid * RPW
    pltpu.sync_copy(idx_hbm.at[pl.ds(base, RPW)], ridx)

    @pl.when(lax.axis_index("s") == 0)
    def _load_g():
        pltpu.sync_copy(g_hbm, gsh)

    plsc.subcore_barrier()
    copies = [
        pltpu.async_copy(gsh.at[ridx.at[j]], dest.at[j], sem_g)
        for j in range(RPW)
    ]
    for c in copies:
        c.wait()
    pltpu.sync_copy(dest, out_hbm.at[pl.ds(base, RPW)])


def kernel(cents, conts_negs, V, U):
    cents = cents.astype(jnp.int32)
    conts = conts_negs.astype(jnp.int32)

    g, idx = pl.pallas_call(
        _prep_body,
        out_shape=(
            jax.ShapeDtypeStruct((GBLK * N_VOCAB, 128), jnp.float32),
            jax.ShapeDtypeStruct((NROWS, 128), jnp.int32),
        ),
    )(V.T, U.T, cents, conts.T)

    logits = _gather_sc(idx, g.reshape(GBLK * N_VOCAB * 128))

    out = pl.pallas_call(
        _softmax_body,
        out_shape=jax.ShapeDtypeStruct((1, 1), jnp.float32),
        out_specs=pl.BlockSpec(memory_space=pltpu.SMEM),
    )(logits)
    return out[0, 0]


# speedup vs baseline: 1.0049x; 1.0049x over previous
import functools

import jax
import jax.numpy as jnp
from jax import lax
from jax.experimental import pallas as pl
from jax.experimental.pallas import tpu as pltpu
from jax.experimental.pallas import tpu_sc as plsc

N_VOCAB = 1000
EMB = 64
B = 16384
KP1 = 11
NPAIR = KP1 * B
NROWS = NPAIR // 128
GBLK = 8

_info = plsc.get_sparse_core_info()
_NC, _NS = _info.num_cores, _info.num_subcores
NW = _NC * _NS
RPW = NROWS // NW


def _prep_body(vt_ref, ut_ref, c_ref, xt_ref, g_ref, idx_ref):
    vt = vt_ref[...]
    for t in range(GBLK):
        hi = min(N_VOCAB, (t + 1) * 128)
        gt = lax.dot_general(
            vt, ut_ref[:, t * 128:hi],
            dimension_numbers=(((0,), (0,)), ((), ())),
            preferred_element_type=jnp.float32)
        if hi - t * 128 < 128:
            gt = jnp.concatenate(
                [gt, jnp.zeros((N_VOCAB, 128 - (hi - t * 128)), jnp.float32)],
                axis=1)
        g_ref[t * N_VOCAB:(t + 1) * N_VOCAB, :] = gt
    xt = xt_ref[...]
    c = c_ref[...]
    y = (xt >> 7) * 128000 + c[None, :] * 128 + (xt & 127)
    idx_ref[...] = y.reshape(NROWS, 128)


def _softmax_body(x_ref, o_ref):
    x = x_ref[...]
    blocks = [x[k * 128:(k + 1) * 128, :] for k in range(KP1)]
    m = blocks[0]
    for bl in blocks[1:]:
        m = jnp.maximum(m, bl)
    s = jnp.zeros_like(m)
    for bl in blocks:
        s = s + jnp.exp(bl - m)
    per = m + jnp.log(s) - blocks[0]
    o_ref[0, 0] = jnp.sum(per) / B


_mesh = plsc.VectorSubcoreMesh(core_axis_name="c", subcore_axis_name="s")


@functools.partial(
    pl.kernel,
    mesh=_mesh,
    compiler_params=pltpu.CompilerParams(use_tc_tiling_on_sc=False),
    out_type=jax.ShapeDtypeStruct((NROWS, 128), jnp.float32),
    scratch_types=[
        pltpu.VMEM((RPW, 128), jnp.int32),
        pltpu.VMEM((RPW, 128), jnp.float32),
        pltpu.SemaphoreType.DMA,
    ],
)
def _gather_sc(idx_hbm, g_hbm, out_hbm, ridx, dest, sem_g):
    wid = lax.axis_index("s") * _NC + lax.axis_index("c")
    base = wid * RPW
    pltpu.sync_copy(idx_hbm.at[pl.ds(base, RPW)], ridx)
    copies = [
        pltpu.async_copy(g_hbm.at[ridx.at[j]], dest.at[j], sem_g)
        for j in range(RPW)
    ]
    for c in copies:
        c.wait()
    pltpu.sync_copy(dest, out_hbm.at[pl.ds(base, RPW)])


def kernel(cents, conts_negs, V, U):
    cents = cents.astype(jnp.int32)
    conts = conts_negs.astype(jnp.int32)

    g, idx = pl.pallas_call(
        _prep_body,
        out_shape=(
            jax.ShapeDtypeStruct((GBLK * N_VOCAB, 128), jnp.float32),
            jax.ShapeDtypeStruct((NROWS, 128), jnp.int32),
        ),
    )(V.T, U.T, cents, conts.T)

    logits = _gather_sc(idx, g.reshape(GBLK * N_VOCAB * 128))

    out = pl.pallas_call(
        _softmax_body,
        out_shape=jax.ShapeDtypeStruct((1, 1), jnp.float32),
        out_specs=pl.BlockSpec(memory_space=pltpu.SMEM),
    )(logits)
    return out[0, 0]
